# baseline (speedup 1.0000x reference)
import jax
import jax.numpy as jnp
import numpy as np
from jax.experimental import pallas as pl
from jax.experimental.pallas import tpu as pltpu

_K_TOP = 1024
_I32_MIN = -2147483648

_C_NP = np.repeat(np.eye(128, dtype=np.float32), 4, axis=0)


def _fused_body(x_ref, kl_ref, c_ref, ct_ref, o_ref):
    z = jnp.zeros((128, 512), jnp.float32)
    for c in range(3):
        for ky in range(4):
            xb = x_ref[0, c, :, ky * 512:(ky + 1) * 512].astype(jnp.bfloat16).astype(jnp.float32)
            kb = kl_ref[c * 4 + ky:c * 4 + ky + 1, :].astype(jnp.float32)
            z = z + xb * kb
    g = jnp.dot(z, c_ref[...], preferred_element_type=jnp.float32,
                precision=jax.lax.Precision.HIGHEST)

    bits = jax.lax.bitcast_convert_type(g, jnp.int32)
    keys = jnp.where(bits >= 0, bits, bits ^ 0x7FFFFFFF)[None]

    jvec = jax.lax.broadcasted_iota(jnp.int32, (16, 1, 1), 0)
    lo = jnp.int32(_I32_MIN)
    for sh in (28, 24, 20, 16, 12, 8, 4, 0):
        cuts = lo + (jvec << sh)
        cnt = jnp.sum((keys >= cuts).astype(jnp.int32), axis=(1, 2),
                      keepdims=True)
        lo = jnp.max(jnp.where(cnt >= _K_TOP, cuts, _I32_MIN))

    m = jnp.where(keys[0] >= lo, g, 0.0)
    m_up = jnp.dot(m, ct_ref[...], preferred_element_type=jnp.float32,
                   precision=jax.lax.Precision.HIGHEST)
    for c in range(3):
        for ky in range(4):
            sl = pl.ds(ky * 512, 512)
            o_ref[0, c, :, sl] = x_ref[0, c, :, sl] * m_up


@jax.jit
def _run(x, kl, cmat, ctmat):
    return pl.pallas_call(
        _fused_body,
        grid=(16,),
        in_specs=[
            pl.BlockSpec((1, 3, 128, 2048), lambda b: (b, 0, 0, 0)),
            pl.BlockSpec((12, 512), lambda b: (0, 0)),
            pl.BlockSpec((512, 128), lambda b: (0, 0)),
            pl.BlockSpec((128, 512), lambda b: (0, 0)),
        ],
        out_specs=pl.BlockSpec((1, 3, 128, 2048), lambda b: (b, 0, 0, 0)),
        out_shape=jax.ShapeDtypeStruct((16, 3, 128, 2048), jnp.float32),
        compiler_params=pltpu.CompilerParams(
            dimension_semantics=("arbitrary",),
        ),
    )(x, kl, cmat, ctmat)


def kernel(inputs, gating_kernel):
    b, cin, H, W = inputs.shape
    x = inputs.reshape(b, cin, 128, 4, 512).reshape(b, cin, 128, 2048)
    w = gating_kernel[0].astype(jnp.bfloat16)
    kl = jnp.tile(w.reshape(12, 1, 4), (1, 128, 1)).reshape(12, 512)
    cmat = jnp.asarray(_C_NP)
    out = _run(x, kl, cmat, cmat.T)
    return out.reshape(b, cin, 128, 4, 512).reshape(b, cin, H, W)
